# Initial kernel scaffold; baseline (speedup 1.0000x reference)
import math

import jax
import jax.numpy as jnp
from jax.experimental import pallas as pl

_EMBD = 128
_SUB = 32
_HALF = 16
_N_DOF = 23


def _body(ang_ref, f_ref, h_ref, d_ref, ht_ref, dt_ref, st_ref, out_ref):
    bb, s = ang_ref.shape

    k = jnp.arange(1, _HALF + 1, dtype=jnp.float32)
    mult = jnp.concatenate([k, k])
    off = jnp.concatenate(
        [jnp.zeros((_HALF,), jnp.float32),
         jnp.full((_HALF,), 0.5 * math.pi, jnp.float32)])
    a = ang_ref[...][..., None]
    seg_angle = jnp.sin(a * mult + off)

    j = jnp.arange(_HALF, dtype=jnp.float32)
    inv_scale = jnp.exp(-(2.0 / _SUB) * math.log(10000.0) * j)
    inv2 = jnp.concatenate([inv_scale, inv_scale]) * math.pi
    fr = f_ref[...].astype(jnp.float32)[..., None]
    seg_frame = jnp.sin(fr * inv2 + off)

    h = h_ref[...][..., None]
    seg_hand = jnp.where(h == 0, ht_ref[0, :], ht_ref[1, :])

    d = d_ref[...][..., None]
    oh = (d == jnp.arange(_N_DOF, dtype=jnp.int32)).astype(jnp.float32)
    seg_dof = jax.lax.dot_general(
        oh.reshape(bb * s, _N_DOF), dt_ref[...],
        (((1,), (0,)), ((), ())),
        preferred_element_type=jnp.float32).reshape(bb, s, _SUB)

    full = jnp.concatenate([seg_angle, seg_hand, seg_frame, seg_dof], axis=-1)
    out_ref[:, 1:, :] = full
    out_ref[:, 0, :] = jnp.broadcast_to(st_ref[0, :], (bb, _EMBD))


def kernel(angles, frame_idxs, hand_idxs, dof_idxs, hand_table, dof_table,
           sentinel_table):
    b, s = angles.shape
    f = frame_idxs.astype(jnp.int32)
    h = hand_idxs.astype(jnp.int32)
    d = dof_idxs.astype(jnp.int32)

    bb = 32
    grid = (b // bb,)
    row = pl.BlockSpec((bb, s), lambda i: (i, 0))
    rep2 = pl.BlockSpec(hand_table.shape, lambda i: (0, 0))
    rep3 = pl.BlockSpec(dof_table.shape, lambda i: (0, 0))
    rep4 = pl.BlockSpec(sentinel_table.shape, lambda i: (0, 0))
    return pl.pallas_call(
        _body,
        grid=grid,
        in_specs=[row, row, row, row, rep2, rep3, rep4],
        out_specs=pl.BlockSpec((bb, s + 1, _EMBD), lambda i: (i, 0, 0)),
        out_shape=jax.ShapeDtypeStruct((b, s + 1, _EMBD), jnp.float32),
    )(angles, f, h, d, hand_table, dof_table, sentinel_table)


# speedup vs baseline: 13.8008x; 13.8008x over previous
import math

import jax
import jax.numpy as jnp
import numpy as np
from jax.experimental import pallas as pl
from jax.experimental.pallas import tpu as pltpu

_EMBD = 128
_SUB = 32
_HALF = 16
_N_HANDS = 2
_N_DOF = 23

_S0 = 6.27973012
_S1 = -41.13623479
_S2 = 78.32684839
_S3 = -57.1154045


def _sin_turns(u):
    r = u - jnp.round(u)
    r2 = r * r
    p = jnp.float32(_S3)
    p = p * r2 + jnp.float32(_S2)
    p = p * r2 + jnp.float32(_S1)
    p = p * r2 + jnp.float32(_S0)
    return p * r


def _phase_consts():
    k = np.arange(1, _HALF + 1, dtype=np.float64)
    c = np.zeros((8, _EMBD), np.float32)
    c[0, 0:_SUB] = (np.concatenate([k, k]) / (2.0 * math.pi)).astype(
        np.float32)
    inv_scale = np.exp(-(2.0 / _SUB) * math.log(10000.0)
                       * np.arange(_HALF, dtype=np.float64))
    c[0, 2 * _SUB:3 * _SUB] = (0.5 * np.concatenate(
        [inv_scale, inv_scale])).astype(np.float32)
    c[1, _HALF:_SUB] = 0.25
    c[1, 2 * _SUB + _HALF:3 * _SUB] = 0.25
    return c


def _body(ang_ref, f_ref, h_ref, d_ref, p_ref, c_ref, st_ref, out_ref):
    bb, s = ang_ref.shape
    li = jax.lax.broadcasted_iota(jnp.int32, (bb, s, _EMBD), 2)

    sm = c_ref[0:1, :]
    so = c_ref[1:2, :]
    a_b = jnp.broadcast_to(ang_ref[...][..., None], (bb, s, _EMBD))
    f_b = jnp.broadcast_to(f_ref[...].astype(jnp.float32)[..., None],
                           (bb, s, _EMBD))
    u = jnp.where(li < 2 * _SUB, a_b, f_b) * sm + so

    cidx = h_ref[...] * _N_DOF + d_ref[...]
    c_b = jnp.broadcast_to(cidx[..., None], (bb, s, _EMBD))
    oh = (li == c_b).astype(jnp.float32)
    m = jax.lax.dot_general(
        oh.reshape(bb * s, _EMBD), p_ref[...],
        (((1,), (0,)), ((), ())),
        preferred_element_type=jnp.float32).reshape(bb, s, _EMBD)

    is_phase = (li < _SUB) | ((li >= 2 * _SUB) & (li < 3 * _SUB))
    vals = jnp.where(is_phase, _sin_turns(u), m)

    out_ref[:, 1:, :] = vals
    out_ref[:, 0, :] = jnp.broadcast_to(st_ref[0, :], (bb, _EMBD))


def _packing_matrix(hand_table, dof_table):
    n_combo = _N_HANDS * _N_DOF
    p = jnp.zeros((_EMBD, _EMBD), jnp.float32)
    hand_rep = jnp.repeat(hand_table, _N_DOF, axis=0)
    dof_rep = jnp.tile(dof_table, (_N_HANDS, 1))
    p = p.at[0:n_combo, _SUB:2 * _SUB].set(hand_rep)
    p = p.at[0:n_combo, 3 * _SUB:].set(dof_rep)
    return p


def kernel(angles, frame_idxs, hand_idxs, dof_idxs, hand_table, dof_table,
           sentinel_table):
    b, s = angles.shape
    f = frame_idxs.astype(jnp.int32)
    h = hand_idxs.astype(jnp.int32)
    d = dof_idxs.astype(jnp.int32)
    p = _packing_matrix(hand_table, dof_table)
    c = jnp.asarray(_phase_consts())

    bb = 64
    grid = (b // bb,)
    row = pl.BlockSpec((bb, s), lambda i: (i, 0))
    rep_p = pl.BlockSpec((_EMBD, _EMBD), lambda i: (0, 0))
    rep_c = pl.BlockSpec((8, _EMBD), lambda i: (0, 0))
    rep_s = pl.BlockSpec(sentinel_table.shape, lambda i: (0, 0))
    return pl.pallas_call(
        _body,
        grid=grid,
        in_specs=[row, row, row, row, rep_p, rep_c, rep_s],
        out_specs=pl.BlockSpec((bb, s + 1, _EMBD), lambda i: (i, 0, 0)),
        out_shape=jax.ShapeDtypeStruct((b, s + 1, _EMBD), jnp.float32),
    )(angles, f, h, d, p, c, sentinel_table)
